# baseline (speedup 1.0000x reference)
import functools

import jax
import jax.numpy as jnp
from jax import lax
from jax.experimental import pallas as pl
from jax.experimental.pallas import tpu as pltpu
from jax.experimental.pallas import tpu_sc as plsc

N_NODES = 10000
N_PAD = 10240
N_EDGES = 320000
CH = 128

NC = 2
NS = 16
NW = NC * NS
E_PER_W = N_EDGES // NW
CHUNK = 80
N_CHUNKS = E_PER_W // CHUNK
ROWS_PER_TILE = N_PAD // NS
DEG_W = 128

_sc_mesh = plsc.VectorSubcoreMesh(core_axis_name="c", subcore_axis_name="s")


@functools.partial(
    pl.kernel,
    mesh=_sc_mesh,
    out_type=jax.ShapeDtypeStruct((NC, N_PAD, DEG_W), jnp.float32),
    scratch_types=[
        pltpu.VMEM((N_CHUNKS, 1, CHUNK), jnp.int32),
        pltpu.VMEM((CHUNK, DEG_W), jnp.float32),
        pltpu.VMEM_SHARED((N_PAD, DEG_W), jnp.float32),
        pltpu.SemaphoreType.DMA,
    ],
)
def _deg_kernel(dst_hbm, ones_hbm, zeros_hbm, out_hbm, dst_v, ones_v, acc, sem):
    c = lax.axis_index("c")
    s = lax.axis_index("s")
    wid = c * NS + s
    pltpu.sync_copy(dst_hbm.at[wid], dst_v)
    pltpu.sync_copy(ones_hbm, ones_v)
    pltpu.sync_copy(zeros_hbm.at[pl.ds(s * ROWS_PER_TILE, ROWS_PER_TILE)],
                    acc.at[pl.ds(s * ROWS_PER_TILE, ROWS_PER_TILE)])
    plsc.subcore_barrier()

    def fire(j, _):
        pltpu.async_copy(ones_v, acc.at[dst_v.at[j, 0]], sem, add=True)
        return 0

    def drain(j, _):
        pltpu.make_async_copy(ones_v, acc.at[dst_v.at[0, 0]], sem).wait()
        return 0

    lax.fori_loop(0, N_CHUNKS, fire, 0)
    lax.fori_loop(0, N_CHUNKS, drain, 0)
    plsc.subcore_barrier()
    pltpu.sync_copy(acc.at[pl.ds(s * ROWS_PER_TILE, ROWS_PER_TILE)],
                    out_hbm.at[c, pl.ds(s * ROWS_PER_TILE, ROWS_PER_TILE)])


def _scale_body(deg0_ref, deg1_ref, x_ref, xs_ref):
    deg = 1.0 + deg0_ref[...][:, 0] + deg1_ref[...][:, 0]
    dinv = 1.0 / jnp.sqrt(deg)
    xs_ref[...] = x_ref[...] * dinv[:, None]


_R2 = 2048

_xs_call = pl.pallas_call(
    _scale_body,
    grid=(N_PAD // _R2,),
    in_specs=[
        pl.BlockSpec((_R2, DEG_W), lambda i: (i, 0)),
        pl.BlockSpec((_R2, DEG_W), lambda i: (i, 0)),
        pl.BlockSpec((_R2, CH), lambda i: (i, 0)),
    ],
    out_specs=pl.BlockSpec((_R2, CH), lambda i: (i, 0)),
    out_shape=jax.ShapeDtypeStruct((N_PAD, CH), jnp.float32),
)


@functools.partial(
    pl.kernel,
    mesh=_sc_mesh,
    out_type=jax.ShapeDtypeStruct((NC, N_PAD, CH), jnp.float32),
    scratch_types=[
        pltpu.VMEM((E_PER_W,), jnp.int32),
        pltpu.VMEM((N_CHUNKS, 1, CHUNK), jnp.int32),
        pltpu.VMEM((CHUNK, CH), jnp.float32),
        pltpu.VMEM((CHUNK, CH), jnp.float32),
        pltpu.VMEM_SHARED((N_PAD, CH), jnp.float32),
        pltpu.SemaphoreType.DMA,
        pltpu.SemaphoreType.DMA,
        pltpu.SemaphoreType.DMA,
        pltpu.SemaphoreType.DMA,
    ],
)
def _agg_kernel(xs_hbm, src_hbm, dst_hbm, zeros_hbm, out_hbm,
                src_v, dst_v, rows0, rows1, acc, gsem0, gsem1, ssem0, ssem1):
    c = lax.axis_index("c")
    s = lax.axis_index("s")
    wid = c * NS + s
    pltpu.sync_copy(src_hbm.at[wid], src_v)
    pltpu.sync_copy(dst_hbm.at[wid], dst_v)
    pltpu.sync_copy(zeros_hbm.at[pl.ds(s * ROWS_PER_TILE, ROWS_PER_TILE)],
                    acc.at[pl.ds(s * ROWS_PER_TILE, ROWS_PER_TILE)])
    plsc.subcore_barrier()

    def g_start(j, buf, sem):
        pltpu.async_copy(
            xs_hbm.at[src_v.at[pl.ds(j * CHUNK, CHUNK)]], buf, sem)

    def g_wait(buf, sem):
        pltpu.make_async_copy(
            xs_hbm.at[src_v.at[pl.ds(0, CHUNK)]], buf, sem).wait()

    def s_start(j, buf, sem):
        pltpu.async_copy(buf, acc.at[dst_v.at[j, 0]], sem, add=True)

    def s_wait(buf, sem):
        pltpu.make_async_copy(buf, acc.at[dst_v.at[0, 0]], sem).wait()

    g_start(0, rows0, gsem0)
    g_start(1, rows1, gsem1)

    def loop_body(j, _):
        g_wait(rows0, gsem0)
        s_start(j, rows0, ssem0)
        g_wait(rows1, gsem1)
        s_start(j + 1, rows1, ssem1)
        s_wait(rows0, ssem0)
        g_start(j + 2, rows0, gsem0)
        s_wait(rows1, ssem1)
        g_start(j + 3, rows1, gsem1)
        return 0

    lax.fori_loop(0, (N_CHUNKS - 3) // 2, lambda i, v: loop_body(2 * i, v), 0)
    g_wait(rows0, gsem0)
    s_start(N_CHUNKS - 3, rows0, ssem0)
    g_wait(rows1, gsem1)
    s_start(N_CHUNKS - 2, rows1, ssem1)
    s_wait(rows0, ssem0)
    g_start(N_CHUNKS - 1, rows0, gsem0)
    g_wait(rows0, gsem0)
    s_start(N_CHUNKS - 1, rows0, ssem0)
    s_wait(rows1, ssem1)
    s_wait(rows0, ssem0)

    plsc.subcore_barrier()
    pltpu.sync_copy(acc.at[pl.ds(s * ROWS_PER_TILE, ROWS_PER_TILE)],
                    out_hbm.at[c, pl.ds(s * ROWS_PER_TILE, ROWS_PER_TILE)])


def _gru_body(deg0_ref, deg1_ref, p0_ref, p1_ref, xs_ref, wz_ref, bz_ref,
              wh_ref, bh_ref, out_ref):
    deg = 1.0 + deg0_ref[...][:, 0] + deg1_ref[...][:, 0]
    dinv = 1.0 / jnp.sqrt(deg)
    p = (p0_ref[...] + p1_ref[...] + xs_ref[...]) * dinv[:, None]
    z = jax.nn.sigmoid(
        jnp.dot(p, wz_ref[...], preferred_element_type=jnp.float32) + bz_ref[...])
    ht = jnp.tanh(
        jnp.dot(p, wh_ref[...], preferred_element_type=jnp.float32) + bh_ref[...])
    out_ref[...] = (1.0 - z) * ht


_R4 = 2048

_gru_call = pl.pallas_call(
    _gru_body,
    grid=(N_PAD // _R4,),
    in_specs=[
        pl.BlockSpec((_R4, DEG_W), lambda i: (i, 0)),
        pl.BlockSpec((_R4, DEG_W), lambda i: (i, 0)),
        pl.BlockSpec((_R4, CH), lambda i: (i, 0)),
        pl.BlockSpec((_R4, CH), lambda i: (i, 0)),
        pl.BlockSpec((_R4, CH), lambda i: (i, 0)),
        pl.BlockSpec((CH, CH), lambda i: (0, 0)),
        pl.BlockSpec((1, CH), lambda i: (0, 0)),
        pl.BlockSpec((CH, CH), lambda i: (0, 0)),
        pl.BlockSpec((1, CH), lambda i: (0, 0)),
    ],
    out_specs=pl.BlockSpec((_R4, CH), lambda i: (i, 0)),
    out_shape=jax.ShapeDtypeStruct((N_PAD, CH), jnp.float32),
)


def kernel(x, edge_index, Wz, bz, Wr, br, Wh, bh):
    src = edge_index[0].reshape(NW, E_PER_W)
    dst = edge_index[1].reshape(NW, N_CHUNKS, 1, CHUNK)

    ones_deg = jnp.ones((CHUNK, DEG_W), jnp.float32)
    zeros_deg = jnp.zeros((N_PAD, DEG_W), jnp.float32)
    zeros_rows = jnp.zeros((N_PAD, CH), jnp.float32)
    x_pad = jnp.pad(x, ((0, N_PAD - N_NODES), (0, 0)))

    deg_parts = _deg_kernel(dst, ones_deg, zeros_deg)
    deg0, deg1 = deg_parts[0], deg_parts[1]
    xs = _xs_call(deg0, deg1, x_pad)
    parts = _agg_kernel(xs, src, dst, zeros_rows)
    out = _gru_call(deg0, deg1, parts[0], parts[1], xs,
                    Wz[:CH], bz.reshape(1, CH), Wh[:CH], bh.reshape(1, CH))
    return out[:N_NODES]


# speedup vs baseline: 1.1326x; 1.1326x over previous
import functools

import jax
import jax.numpy as jnp
from jax import lax
from jax.experimental import pallas as pl
from jax.experimental.pallas import tpu as pltpu
from jax.experimental.pallas import tpu_sc as plsc

N_NODES = 10000
N_PAD = 10240
N_EDGES = 320000
CH = 128

NC = 2
NS = 16
NW = NC * NS
E_PER_W = N_EDGES // NW
CHUNK = 80
N_CHUNKS = E_PER_W // CHUNK
ROWS_PER_TILE = N_PAD // NS
DEG_W = 128

_sc_mesh = plsc.VectorSubcoreMesh(core_axis_name="c", subcore_axis_name="s")


@functools.partial(
    pl.kernel,
    mesh=_sc_mesh,
    out_type=jax.ShapeDtypeStruct((NC, N_PAD, DEG_W), jnp.float32),
    scratch_types=[
        pltpu.VMEM((N_CHUNKS, 1, CHUNK), jnp.int32),
        pltpu.VMEM((CHUNK, DEG_W), jnp.float32),
        pltpu.VMEM_SHARED((N_PAD, DEG_W), jnp.float32),
        pltpu.SemaphoreType.DMA,
    ],
)
def _deg_kernel(dst_hbm, ones_hbm, zeros_hbm, out_hbm, dst_v, ones_v, acc, sem):
    c = lax.axis_index("c")
    s = lax.axis_index("s")
    wid = c * NS + s
    pltpu.sync_copy(dst_hbm.at[wid], dst_v)
    pltpu.sync_copy(ones_hbm, ones_v)
    pltpu.sync_copy(zeros_hbm.at[pl.ds(s * ROWS_PER_TILE, ROWS_PER_TILE)],
                    acc.at[pl.ds(s * ROWS_PER_TILE, ROWS_PER_TILE)])
    plsc.subcore_barrier()

    def fire(j, _):
        pltpu.async_copy(ones_v, acc.at[dst_v.at[j, 0]], sem, add=True)
        return 0

    def drain(j, _):
        pltpu.make_async_copy(ones_v, acc.at[dst_v.at[0, 0]], sem).wait()
        return 0

    lax.fori_loop(0, N_CHUNKS, fire, 0)
    lax.fori_loop(0, N_CHUNKS, drain, 0)
    plsc.subcore_barrier()
    pltpu.sync_copy(acc.at[pl.ds(s * ROWS_PER_TILE, ROWS_PER_TILE)],
                    out_hbm.at[c, pl.ds(s * ROWS_PER_TILE, ROWS_PER_TILE)])


def _scale_body(deg0_ref, deg1_ref, x_ref, xs_ref):
    deg = 1.0 + deg0_ref[...][:, 0] + deg1_ref[...][:, 0]
    dinv = 1.0 / jnp.sqrt(deg)
    xs_ref[...] = x_ref[...] * dinv[:, None]


_R2 = 2048

_xs_call = pl.pallas_call(
    _scale_body,
    grid=(N_PAD // _R2,),
    in_specs=[
        pl.BlockSpec((_R2, DEG_W), lambda i: (i, 0)),
        pl.BlockSpec((_R2, DEG_W), lambda i: (i, 0)),
        pl.BlockSpec((_R2, CH), lambda i: (i, 0)),
    ],
    out_specs=pl.BlockSpec((_R2, CH), lambda i: (i, 0)),
    out_shape=jax.ShapeDtypeStruct((N_PAD, CH), jnp.float32),
)


@functools.partial(
    pl.kernel,
    mesh=_sc_mesh,
    out_type=jax.ShapeDtypeStruct((NC, N_PAD, CH), jnp.float32),
    scratch_types=[
        pltpu.VMEM((E_PER_W,), jnp.int32),
        pltpu.VMEM((N_CHUNKS, 1, CHUNK), jnp.int32),
        pltpu.VMEM((CHUNK, CH), jnp.float32),
        pltpu.VMEM((CHUNK, CH), jnp.float32),
        pltpu.VMEM_SHARED((N_PAD, CH), jnp.float32),
        pltpu.SemaphoreType.DMA,
        pltpu.SemaphoreType.DMA,
        pltpu.SemaphoreType.DMA,
        pltpu.SemaphoreType.DMA,
    ],
)
def _agg_kernel(xs_hbm, src_hbm, dst_hbm, zeros_hbm, out_hbm,
                src_v, dst_v, rows0, rows1, acc, gsem0, gsem1, ssem0, ssem1):
    c = lax.axis_index("c")
    s = lax.axis_index("s")
    wid = c * NS + s
    pltpu.sync_copy(src_hbm.at[wid], src_v)
    pltpu.sync_copy(dst_hbm.at[wid], dst_v)
    pltpu.sync_copy(zeros_hbm.at[pl.ds(s * ROWS_PER_TILE, ROWS_PER_TILE)],
                    acc.at[pl.ds(s * ROWS_PER_TILE, ROWS_PER_TILE)])
    plsc.subcore_barrier()

    def g_start(j, buf, sem):
        pltpu.async_copy(
            xs_hbm.at[src_v.at[pl.ds(j * CHUNK, CHUNK)]], buf, sem)

    def g_wait(buf, sem):
        pltpu.make_async_copy(
            xs_hbm.at[src_v.at[pl.ds(0, CHUNK)]], buf, sem).wait()

    def s_start(j, buf, sem):
        pltpu.async_copy(buf, acc.at[dst_v.at[j, 0]], sem, add=True)

    def s_wait(buf, sem):
        pltpu.make_async_copy(buf, acc.at[dst_v.at[0, 0]], sem).wait()

    g_start(0, rows0, gsem0)
    g_start(1, rows1, gsem1)

    def loop_body(j, _):
        g_wait(rows0, gsem0)
        s_start(j, rows0, ssem0)
        s_wait(rows0, ssem0)
        g_start(j + 2, rows0, gsem0)
        g_wait(rows1, gsem1)
        s_start(j + 1, rows1, ssem1)
        s_wait(rows1, ssem1)
        g_start(j + 3, rows1, gsem1)
        return 0

    lax.fori_loop(0, (N_CHUNKS - 3) // 2, lambda i, v: loop_body(2 * i, v), 0)
    g_wait(rows0, gsem0)
    s_start(N_CHUNKS - 3, rows0, ssem0)
    s_wait(rows0, ssem0)
    g_start(N_CHUNKS - 1, rows0, gsem0)
    g_wait(rows1, gsem1)
    s_start(N_CHUNKS - 2, rows1, ssem1)
    s_wait(rows1, ssem1)
    g_wait(rows0, gsem0)
    s_start(N_CHUNKS - 1, rows0, ssem0)
    s_wait(rows0, ssem0)

    plsc.subcore_barrier()
    pltpu.sync_copy(acc.at[pl.ds(s * ROWS_PER_TILE, ROWS_PER_TILE)],
                    out_hbm.at[c, pl.ds(s * ROWS_PER_TILE, ROWS_PER_TILE)])


def _gru_body(deg0_ref, deg1_ref, p0_ref, p1_ref, xs_ref, wz_ref, bz_ref,
              wh_ref, bh_ref, out_ref):
    deg = 1.0 + deg0_ref[...][:, 0] + deg1_ref[...][:, 0]
    dinv = 1.0 / jnp.sqrt(deg)
    p = (p0_ref[...] + p1_ref[...] + xs_ref[...]) * dinv[:, None]
    z = jax.nn.sigmoid(
        jnp.dot(p, wz_ref[...], preferred_element_type=jnp.float32) + bz_ref[...])
    ht = jnp.tanh(
        jnp.dot(p, wh_ref[...], preferred_element_type=jnp.float32) + bh_ref[...])
    out_ref[...] = (1.0 - z) * ht


_R4 = 2048

_gru_call = pl.pallas_call(
    _gru_body,
    grid=(N_PAD // _R4,),
    in_specs=[
        pl.BlockSpec((_R4, DEG_W), lambda i: (i, 0)),
        pl.BlockSpec((_R4, DEG_W), lambda i: (i, 0)),
        pl.BlockSpec((_R4, CH), lambda i: (i, 0)),
        pl.BlockSpec((_R4, CH), lambda i: (i, 0)),
        pl.BlockSpec((_R4, CH), lambda i: (i, 0)),
        pl.BlockSpec((CH, CH), lambda i: (0, 0)),
        pl.BlockSpec((1, CH), lambda i: (0, 0)),
        pl.BlockSpec((CH, CH), lambda i: (0, 0)),
        pl.BlockSpec((1, CH), lambda i: (0, 0)),
    ],
    out_specs=pl.BlockSpec((_R4, CH), lambda i: (i, 0)),
    out_shape=jax.ShapeDtypeStruct((N_PAD, CH), jnp.float32),
)


def kernel(x, edge_index, Wz, bz, Wr, br, Wh, bh):
    src = edge_index[0].reshape(NW, E_PER_W)
    dst = edge_index[1].reshape(NW, N_CHUNKS, 1, CHUNK)

    ones_deg = jnp.ones((CHUNK, DEG_W), jnp.float32)
    zeros_deg = jnp.zeros((N_PAD, DEG_W), jnp.float32)
    zeros_rows = jnp.zeros((N_PAD, CH), jnp.float32)
    x_pad = jnp.pad(x, ((0, N_PAD - N_NODES), (0, 0)))

    deg_parts = _deg_kernel(dst, ones_deg, zeros_deg)
    deg0, deg1 = deg_parts[0], deg_parts[1]
    xs = _xs_call(deg0, deg1, x_pad)
    parts = _agg_kernel(xs, src, dst, zeros_rows)
    out = _gru_call(deg0, deg1, parts[0], parts[1], xs,
                    Wz[:CH], bz.reshape(1, CH), Wh[:CH], bh.reshape(1, CH))
    return out[:N_NODES]


# speedup vs baseline: 1.1429x; 1.0091x over previous
import functools

import jax
import jax.numpy as jnp
from jax import lax
from jax.experimental import pallas as pl
from jax.experimental.pallas import tpu as pltpu
from jax.experimental.pallas import tpu_sc as plsc

N_NODES = 10000
N_PAD = 10240
N_EDGES = 320000
CH = 128

NC = 2
NS = 16
NW = NC * NS
E_PER_W = N_EDGES // NW
C1 = 128
NCH1 = 10240 // C1
E1_PER_W = NCH1 * C1
CHUNK = 80
N_CHUNKS = E_PER_W // CHUNK
ROWS_PER_TILE = N_PAD // NS
DEG_W = 128

_sc_mesh = plsc.VectorSubcoreMesh(core_axis_name="c", subcore_axis_name="s")


@functools.partial(
    pl.kernel,
    mesh=_sc_mesh,
    out_type=jax.ShapeDtypeStruct((NC, N_PAD, DEG_W), jnp.float32),
    scratch_types=[
        pltpu.VMEM((NCH1, 1, C1), jnp.int32),
        pltpu.VMEM((C1, DEG_W), jnp.float32),
        pltpu.VMEM_SHARED((N_PAD, DEG_W), jnp.float32),
        pltpu.SemaphoreType.DMA,
    ],
)
def _deg_kernel(dst_hbm, ones_hbm, zeros_hbm, out_hbm, dst_v, ones_v, acc, sem):
    c = lax.axis_index("c")
    s = lax.axis_index("s")
    wid = c * NS + s
    pltpu.sync_copy(dst_hbm.at[wid], dst_v)
    pltpu.sync_copy(ones_hbm, ones_v)
    pltpu.sync_copy(zeros_hbm.at[pl.ds(s * ROWS_PER_TILE, ROWS_PER_TILE)],
                    acc.at[pl.ds(s * ROWS_PER_TILE, ROWS_PER_TILE)])
    plsc.subcore_barrier()

    def fire(j, _):
        pltpu.async_copy(ones_v, acc.at[dst_v.at[j, 0]], sem, add=True)
        return 0

    def drain(j, _):
        pltpu.make_async_copy(ones_v, acc.at[dst_v.at[0, 0]], sem).wait()
        return 0

    lax.fori_loop(0, NCH1, fire, 0)
    lax.fori_loop(0, NCH1, drain, 0)
    plsc.subcore_barrier()
    pltpu.sync_copy(acc.at[pl.ds(s * ROWS_PER_TILE, ROWS_PER_TILE)],
                    out_hbm.at[c, pl.ds(s * ROWS_PER_TILE, ROWS_PER_TILE)])


def _scale_body(deg0_ref, deg1_ref, x_ref, xs_ref):
    deg = 1.0 + deg0_ref[...][:, 0] + deg1_ref[...][:, 0]
    dinv = 1.0 / jnp.sqrt(deg)
    xs_ref[...] = x_ref[...] * dinv[:, None]


_R2 = 2048

_xs_call = pl.pallas_call(
    _scale_body,
    grid=(N_PAD // _R2,),
    in_specs=[
        pl.BlockSpec((_R2, DEG_W), lambda i: (i, 0)),
        pl.BlockSpec((_R2, DEG_W), lambda i: (i, 0)),
        pl.BlockSpec((_R2, CH), lambda i: (i, 0)),
    ],
    out_specs=pl.BlockSpec((_R2, CH), lambda i: (i, 0)),
    out_shape=jax.ShapeDtypeStruct((N_PAD, CH), jnp.float32),
)


@functools.partial(
    pl.kernel,
    mesh=_sc_mesh,
    out_type=jax.ShapeDtypeStruct((NC, N_PAD, CH), jnp.float32),
    scratch_types=[
        pltpu.VMEM((E_PER_W,), jnp.int32),
        pltpu.VMEM((N_CHUNKS, 1, CHUNK), jnp.int32),
        pltpu.VMEM((CHUNK, CH), jnp.float32),
        pltpu.VMEM((CHUNK, CH), jnp.float32),
        pltpu.VMEM_SHARED((N_PAD, CH), jnp.float32),
        pltpu.SemaphoreType.DMA,
        pltpu.SemaphoreType.DMA,
        pltpu.SemaphoreType.DMA,
        pltpu.SemaphoreType.DMA,
    ],
)
def _agg_kernel(xs_hbm, src_hbm, dst_hbm, zeros_hbm, out_hbm,
                src_v, dst_v, rows0, rows1, acc, gsem0, gsem1, ssem0, ssem1):
    c = lax.axis_index("c")
    s = lax.axis_index("s")
    wid = c * NS + s
    pltpu.sync_copy(src_hbm.at[wid], src_v)
    pltpu.sync_copy(dst_hbm.at[wid], dst_v)
    pltpu.sync_copy(zeros_hbm.at[pl.ds(s * ROWS_PER_TILE, ROWS_PER_TILE)],
                    acc.at[pl.ds(s * ROWS_PER_TILE, ROWS_PER_TILE)])
    plsc.subcore_barrier()

    def g_start(j, buf, sem):
        pltpu.async_copy(
            xs_hbm.at[src_v.at[pl.ds(j * CHUNK, CHUNK)]], buf, sem)

    def g_wait(buf, sem):
        pltpu.make_async_copy(
            xs_hbm.at[src_v.at[pl.ds(0, CHUNK)]], buf, sem).wait()

    def s_start(j, buf, sem):
        pltpu.async_copy(buf, acc.at[dst_v.at[j, 0]], sem, add=True)

    def s_wait(buf, sem):
        pltpu.make_async_copy(buf, acc.at[dst_v.at[0, 0]], sem).wait()

    g_start(0, rows0, gsem0)
    g_start(1, rows1, gsem1)

    def loop_body(j, _):
        g_wait(rows0, gsem0)
        s_start(j, rows0, ssem0)
        s_wait(rows0, ssem0)
        g_start(j + 2, rows0, gsem0)
        g_wait(rows1, gsem1)
        s_start(j + 1, rows1, ssem1)
        s_wait(rows1, ssem1)
        g_start(j + 3, rows1, gsem1)
        return 0

    lax.fori_loop(0, (N_CHUNKS - 3) // 2, lambda i, v: loop_body(2 * i, v), 0)
    g_wait(rows0, gsem0)
    s_start(N_CHUNKS - 3, rows0, ssem0)
    s_wait(rows0, ssem0)
    g_start(N_CHUNKS - 1, rows0, gsem0)
    g_wait(rows1, gsem1)
    s_start(N_CHUNKS - 2, rows1, ssem1)
    s_wait(rows1, ssem1)
    g_wait(rows0, gsem0)
    s_start(N_CHUNKS - 1, rows0, ssem0)
    s_wait(rows0, ssem0)

    plsc.subcore_barrier()
    pltpu.sync_copy(acc.at[pl.ds(s * ROWS_PER_TILE, ROWS_PER_TILE)],
                    out_hbm.at[c, pl.ds(s * ROWS_PER_TILE, ROWS_PER_TILE)])


def _gru_body(deg0_ref, deg1_ref, p0_ref, p1_ref, xs_ref, wz_ref, bz_ref,
              wh_ref, bh_ref, out_ref):
    deg = 1.0 + deg0_ref[...][:, 0] + deg1_ref[...][:, 0]
    dinv = 1.0 / jnp.sqrt(deg)
    p = (p0_ref[...] + p1_ref[...] + xs_ref[...]) * dinv[:, None]
    z = jax.nn.sigmoid(
        jnp.dot(p, wz_ref[...], preferred_element_type=jnp.float32) + bz_ref[...])
    ht = jnp.tanh(
        jnp.dot(p, wh_ref[...], preferred_element_type=jnp.float32) + bh_ref[...])
    out_ref[...] = (1.0 - z) * ht


_R4 = 2048

_gru_call = pl.pallas_call(
    _gru_body,
    grid=(N_PAD // _R4,),
    in_specs=[
        pl.BlockSpec((_R4, DEG_W), lambda i: (i, 0)),
        pl.BlockSpec((_R4, DEG_W), lambda i: (i, 0)),
        pl.BlockSpec((_R4, CH), lambda i: (i, 0)),
        pl.BlockSpec((_R4, CH), lambda i: (i, 0)),
        pl.BlockSpec((_R4, CH), lambda i: (i, 0)),
        pl.BlockSpec((CH, CH), lambda i: (0, 0)),
        pl.BlockSpec((1, CH), lambda i: (0, 0)),
        pl.BlockSpec((CH, CH), lambda i: (0, 0)),
        pl.BlockSpec((1, CH), lambda i: (0, 0)),
    ],
    out_specs=pl.BlockSpec((_R4, CH), lambda i: (i, 0)),
    out_shape=jax.ShapeDtypeStruct((N_PAD, CH), jnp.float32),
)


def kernel(x, edge_index, Wz, bz, Wr, br, Wh, bh):
    src = edge_index[0].reshape(NW, E_PER_W)
    dst = edge_index[1].reshape(NW, N_CHUNKS, 1, CHUNK)
    pad = jnp.full((NW, E1_PER_W - E_PER_W), N_PAD - 1, edge_index.dtype)
    dst1 = jnp.concatenate([edge_index[1].reshape(NW, E_PER_W), pad], axis=1)
    dst1 = dst1.reshape(NW, NCH1, 1, C1)

    ones_deg = jnp.ones((C1, DEG_W), jnp.float32)
    zeros_deg = jnp.zeros((N_PAD, DEG_W), jnp.float32)
    zeros_rows = jnp.zeros((N_PAD, CH), jnp.float32)
    x_pad = jnp.pad(x, ((0, N_PAD - N_NODES), (0, 0)))

    deg_parts = _deg_kernel(dst1, ones_deg, zeros_deg)
    deg0, deg1 = deg_parts[0], deg_parts[1]
    xs = _xs_call(deg0, deg1, x_pad)
    parts = _agg_kernel(xs, src, dst, zeros_rows)
    out = _gru_call(deg0, deg1, parts[0], parts[1], xs,
                    Wz[:CH], bz.reshape(1, CH), Wh[:CH], bh.reshape(1, CH))
    return out[:N_NODES]


# speedup vs baseline: 1.1595x; 1.0146x over previous
import functools

import jax
import jax.numpy as jnp
from jax import lax
from jax.experimental import pallas as pl
from jax.experimental.pallas import tpu as pltpu
from jax.experimental.pallas import tpu_sc as plsc

N_NODES = 10000
N_PAD = 10240
N_EDGES = 320000
CH = 128

NC = 2
NS = 16
NW = NC * NS
E_PER_W = N_EDGES // NW
C1 = 128
NCH1 = 10240 // C1
E1_PER_W = NCH1 * C1
CHUNK = 80
N_CHUNKS = E_PER_W // CHUNK
ROWS_PER_TILE = N_PAD // NS
DEG_W = 128

_sc_mesh = plsc.VectorSubcoreMesh(core_axis_name="c", subcore_axis_name="s")


@functools.partial(
    pl.kernel,
    mesh=_sc_mesh,
    out_type=jax.ShapeDtypeStruct((NC, N_PAD, DEG_W), jnp.float32),
    scratch_types=[
        pltpu.VMEM((NCH1, 1, C1), jnp.int32),
        pltpu.VMEM((C1, DEG_W), jnp.float32),
        pltpu.VMEM_SHARED((N_PAD, DEG_W), jnp.float32),
        pltpu.SemaphoreType.DMA,
    ],
)
def _deg_kernel(dst_hbm, ones_hbm, zeros_hbm, out_hbm, dst_v, ones_v, acc, sem):
    c = lax.axis_index("c")
    s = lax.axis_index("s")
    wid = c * NS + s
    pltpu.sync_copy(dst_hbm.at[wid], dst_v)
    pltpu.sync_copy(ones_hbm, ones_v)
    pltpu.sync_copy(zeros_hbm.at[pl.ds(s * ROWS_PER_TILE, ROWS_PER_TILE)],
                    acc.at[pl.ds(s * ROWS_PER_TILE, ROWS_PER_TILE)])
    plsc.subcore_barrier()

    def fire(j, _):
        pltpu.async_copy(ones_v, acc.at[dst_v.at[j, 0]], sem, add=True)
        return 0

    def drain(j, _):
        pltpu.make_async_copy(ones_v, acc.at[dst_v.at[0, 0]], sem).wait()
        return 0

    lax.fori_loop(0, NCH1, fire, 0)
    lax.fori_loop(0, NCH1, drain, 0)
    plsc.subcore_barrier()
    pltpu.sync_copy(acc.at[pl.ds(s * ROWS_PER_TILE, ROWS_PER_TILE)],
                    out_hbm.at[c, pl.ds(s * ROWS_PER_TILE, ROWS_PER_TILE)])


def _scale_body(deg0_ref, deg1_ref, x_ref, xs_ref):
    deg = 1.0 + deg0_ref[...][:, 0] + deg1_ref[...][:, 0]
    dinv = 1.0 / jnp.sqrt(deg)
    row = pl.program_id(0) * _R2 + lax.broadcasted_iota(jnp.int32, (_R2, 1), 0)
    xs_ref[...] = jnp.where(row < N_NODES, x_ref[...] * dinv[:, None], 0.0)


_R2 = 2048

_xs_call = pl.pallas_call(
    _scale_body,
    grid=(N_PAD // _R2,),
    in_specs=[
        pl.BlockSpec((_R2, DEG_W), lambda i: (i, 0)),
        pl.BlockSpec((_R2, DEG_W), lambda i: (i, 0)),
        pl.BlockSpec((_R2, CH), lambda i: (i, 0)),
    ],
    out_specs=pl.BlockSpec((_R2, CH), lambda i: (i, 0)),
    out_shape=jax.ShapeDtypeStruct((N_PAD, CH), jnp.float32),
)


@functools.partial(
    pl.kernel,
    mesh=_sc_mesh,
    out_type=jax.ShapeDtypeStruct((NC, N_PAD, CH), jnp.float32),
    scratch_types=[
        pltpu.VMEM((E_PER_W,), jnp.int32),
        pltpu.VMEM((N_CHUNKS, 1, CHUNK), jnp.int32),
        pltpu.VMEM((CHUNK, CH), jnp.float32),
        pltpu.VMEM((CHUNK, CH), jnp.float32),
        pltpu.VMEM_SHARED((N_PAD, CH), jnp.float32),
        pltpu.SemaphoreType.DMA,
        pltpu.SemaphoreType.DMA,
        pltpu.SemaphoreType.DMA,
        pltpu.SemaphoreType.DMA,
    ],
)
def _agg_kernel(xs_hbm, src_hbm, dst_hbm, zeros_hbm, out_hbm,
                src_v, dst_v, rows0, rows1, acc, gsem0, gsem1, ssem0, ssem1):
    c = lax.axis_index("c")
    s = lax.axis_index("s")
    wid = c * NS + s
    pltpu.sync_copy(src_hbm.at[wid], src_v)
    pltpu.sync_copy(dst_hbm.at[wid], dst_v)
    pltpu.sync_copy(zeros_hbm.at[pl.ds(s * ROWS_PER_TILE, ROWS_PER_TILE)],
                    acc.at[pl.ds(s * ROWS_PER_TILE, ROWS_PER_TILE)])
    plsc.subcore_barrier()

    def g_start(j, buf, sem):
        pltpu.async_copy(
            xs_hbm.at[src_v.at[pl.ds(j * CHUNK, CHUNK)]], buf, sem)

    def g_wait(buf, sem):
        pltpu.make_async_copy(
            xs_hbm.at[src_v.at[pl.ds(0, CHUNK)]], buf, sem).wait()

    def s_start(j, buf, sem):
        pltpu.async_copy(buf, acc.at[dst_v.at[j, 0]], sem, add=True)

    def s_wait(buf, sem):
        pltpu.make_async_copy(buf, acc.at[dst_v.at[0, 0]], sem).wait()

    g_start(0, rows0, gsem0)
    g_start(1, rows1, gsem1)

    def loop_body(j, _):
        g_wait(rows0, gsem0)
        s_start(j, rows0, ssem0)
        s_wait(rows0, ssem0)
        g_start(j + 2, rows0, gsem0)
        g_wait(rows1, gsem1)
        s_start(j + 1, rows1, ssem1)
        s_wait(rows1, ssem1)
        g_start(j + 3, rows1, gsem1)
        return 0

    lax.fori_loop(0, (N_CHUNKS - 3) // 2, lambda i, v: loop_body(2 * i, v), 0)
    g_wait(rows0, gsem0)
    s_start(N_CHUNKS - 3, rows0, ssem0)
    s_wait(rows0, ssem0)
    g_start(N_CHUNKS - 1, rows0, gsem0)
    g_wait(rows1, gsem1)
    s_start(N_CHUNKS - 2, rows1, ssem1)
    s_wait(rows1, ssem1)
    g_wait(rows0, gsem0)
    s_start(N_CHUNKS - 1, rows0, ssem0)
    s_wait(rows0, ssem0)

    plsc.subcore_barrier()
    pltpu.sync_copy(acc.at[pl.ds(s * ROWS_PER_TILE, ROWS_PER_TILE)],
                    out_hbm.at[c, pl.ds(s * ROWS_PER_TILE, ROWS_PER_TILE)])


def _gru_body(deg0_ref, deg1_ref, p0_ref, p1_ref, xs_ref, wz_ref, bz_ref,
              wh_ref, bh_ref, out_ref):
    deg = 1.0 + deg0_ref[...][:, 0] + deg1_ref[...][:, 0]
    dinv = 1.0 / jnp.sqrt(deg)
    p = (p0_ref[...] + p1_ref[...] + xs_ref[...]) * dinv[:, None]
    z = jax.nn.sigmoid(
        jnp.dot(p, wz_ref[...], preferred_element_type=jnp.float32) + bz_ref[...])
    ht = jnp.tanh(
        jnp.dot(p, wh_ref[...], preferred_element_type=jnp.float32) + bh_ref[...])
    out_ref[...] = (1.0 - z) * ht


_R4 = 2048

_gru_call = pl.pallas_call(
    _gru_body,
    grid=(N_PAD // _R4,),
    in_specs=[
        pl.BlockSpec((_R4, DEG_W), lambda i: (i, 0)),
        pl.BlockSpec((_R4, DEG_W), lambda i: (i, 0)),
        pl.BlockSpec((_R4, CH), lambda i: (i, 0)),
        pl.BlockSpec((_R4, CH), lambda i: (i, 0)),
        pl.BlockSpec((_R4, CH), lambda i: (i, 0)),
        pl.BlockSpec((CH, CH), lambda i: (0, 0)),
        pl.BlockSpec((1, CH), lambda i: (0, 0)),
        pl.BlockSpec((CH, CH), lambda i: (0, 0)),
        pl.BlockSpec((1, CH), lambda i: (0, 0)),
    ],
    out_specs=pl.BlockSpec((_R4, CH), lambda i: (i, 0)),
    out_shape=jax.ShapeDtypeStruct((N_NODES, CH), jnp.float32),
)


def kernel(x, edge_index, Wz, bz, Wr, br, Wh, bh):
    src = edge_index[0].reshape(NW, E_PER_W)
    dst = edge_index[1].reshape(NW, N_CHUNKS, 1, CHUNK)
    pad = jnp.full((NW, E1_PER_W - E_PER_W), N_PAD - 1, edge_index.dtype)
    dst1 = jnp.concatenate([edge_index[1].reshape(NW, E_PER_W), pad], axis=1)
    dst1 = dst1.reshape(NW, NCH1, 1, C1)

    ones_deg = jnp.ones((C1, DEG_W), jnp.float32)
    zeros_deg = jnp.zeros((N_PAD, DEG_W), jnp.float32)
    zeros_rows = jnp.zeros((N_PAD, CH), jnp.float32)

    deg_parts = _deg_kernel(dst1, ones_deg, zeros_deg)
    deg0, deg1 = deg_parts[0], deg_parts[1]
    xs = _xs_call(deg0, deg1, x)
    parts = _agg_kernel(xs, src, dst, zeros_rows)
    return _gru_call(deg0, deg1, parts[0], parts[1], xs,
                     Wz[:CH], bz.reshape(1, CH), Wh[:CH], bh.reshape(1, CH))
